# Initial kernel scaffold; baseline (speedup 1.0000x reference)
import jax
import jax.numpy as jnp
from jax.experimental import pallas as pl


def kernel(x, table):
    raise NotImplementedError("write your pallas kernel here")


# speedup vs baseline: 1.8431x; 1.8431x over previous
import functools

import jax
import jax.numpy as jnp
from jax import lax
from jax.experimental import pallas as pl
from jax.experimental.pallas import tpu as pltpu
from jax.experimental.pallas import tpu_sc as plsc

BATCH = 16384
HIST = 50
EMB_DIM = 64
TOTAL = BATCH * HIST

NUM_CORES = 2
NUM_SUBCORES = 16
NW = NUM_CORES * NUM_SUBCORES
B_PER_W = TOTAL // NW
CHUNK = 1024
N_CHUNKS = B_PER_W // CHUNK


def _gather_body(x_hbm, table_hbm, out_hbm, idx_v, rows_v, sem):
    wid = lax.axis_index("s") * NUM_CORES + lax.axis_index("c")
    base = wid * B_PER_W

    def body(i, carry):
        off = pl.multiple_of(base + i * CHUNK, CHUNK)
        pltpu.sync_copy(x_hbm.at[pl.ds(off, CHUNK)], idx_v)
        pltpu.async_copy(table_hbm.at[idx_v], rows_v, sem).wait()
        pltpu.sync_copy(rows_v, out_hbm.at[pl.ds(off, CHUNK)])
        return carry

    lax.fori_loop(0, N_CHUNKS, body, 0)


@functools.partial(jax.jit, static_argnames=())
def _gather(x_flat, table):
    mesh = plsc.VectorSubcoreMesh(core_axis_name="c", subcore_axis_name="s")
    k = pl.kernel(
        _gather_body,
        out_type=jax.ShapeDtypeStruct((TOTAL, EMB_DIM), jnp.float32),
        mesh=mesh,
        compiler_params=pltpu.CompilerParams(use_tc_tiling_on_sc=False),
        scratch_types=[
            pltpu.VMEM((CHUNK,), jnp.int32),
            pltpu.VMEM((CHUNK, EMB_DIM), jnp.float32),
            pltpu.SemaphoreType.DMA,
        ],
    )
    return k(x_flat, table)


def kernel(x, table):
    x_flat = x.reshape(-1).astype(jnp.int32)
    out = _gather(x_flat, table)
    return out.reshape(BATCH, HIST, EMB_DIM)


# speedup vs baseline: 1.8642x; 1.0114x over previous
import functools

import jax
import jax.numpy as jnp
from jax import lax
from jax.experimental import pallas as pl
from jax.experimental.pallas import tpu as pltpu
from jax.experimental.pallas import tpu_sc as plsc

BATCH = 16384
HIST = 50
EMB_DIM = 64
TOTAL = BATCH * HIST

NUM_CORES = 2
NUM_SUBCORES = 16
NW = NUM_CORES * NUM_SUBCORES
B_PER_W = TOTAL // NW
CHUNK = 640
N_CHUNKS = B_PER_W // CHUNK
N_PAIRS = N_CHUNKS // 2


def _gather_body(x_hbm, table_hbm, out_hbm,
                 idx_v, rows_a, rows_b, sem_ga, sem_gb, sem_sa, sem_sb):
    wid = lax.axis_index("s") * NUM_CORES + lax.axis_index("c")
    base = wid * B_PER_W

    def idx_slice(i):
        return idx_v.at[pl.ds(pl.multiple_of(i * CHUNK, CHUNK), CHUNK)]

    def out_slice(i):
        return out_hbm.at[pl.ds(pl.multiple_of(base + i * CHUNK, CHUNK), CHUNK)]

    pltpu.sync_copy(x_hbm.at[pl.ds(base, B_PER_W)], idx_v)

    pltpu.async_copy(table_hbm.at[idx_slice(0)], rows_a, sem_ga)
    pltpu.async_copy(table_hbm.at[idx_slice(1)], rows_b, sem_gb)

    def body(j, carry):
        i0 = 2 * j
        i1 = i0 + 1
        pltpu.make_async_copy(table_hbm.at[idx_slice(i0)], rows_a, sem_ga).wait()
        pltpu.async_copy(rows_a, out_slice(i0), sem_sa)

        pltpu.make_async_copy(table_hbm.at[idx_slice(i1)], rows_b, sem_gb).wait()
        pltpu.async_copy(rows_b, out_slice(i1), sem_sb)

        @pl.when(j < N_PAIRS - 1)
        def _():
            pltpu.make_async_copy(rows_a, out_slice(i0), sem_sa).wait()
            pltpu.async_copy(table_hbm.at[idx_slice(i0 + 2)], rows_a, sem_ga)
            pltpu.make_async_copy(rows_b, out_slice(i1), sem_sb).wait()
            pltpu.async_copy(table_hbm.at[idx_slice(i1 + 2)], rows_b, sem_gb)

        return carry

    lax.fori_loop(0, N_PAIRS, body, 0)

    pltpu.make_async_copy(rows_a, out_slice(N_CHUNKS - 2), sem_sa).wait()
    pltpu.make_async_copy(rows_b, out_slice(N_CHUNKS - 1), sem_sb).wait()


@jax.jit
def _gather(x_flat, table):
    mesh = plsc.VectorSubcoreMesh(core_axis_name="c", subcore_axis_name="s")
    k = pl.kernel(
        _gather_body,
        out_type=jax.ShapeDtypeStruct((TOTAL, EMB_DIM), jnp.float32),
        mesh=mesh,
        compiler_params=pltpu.CompilerParams(use_tc_tiling_on_sc=False),
        scratch_types=[
            pltpu.VMEM((B_PER_W,), jnp.int32),
            pltpu.VMEM((CHUNK, EMB_DIM), jnp.float32),
            pltpu.VMEM((CHUNK, EMB_DIM), jnp.float32),
            pltpu.SemaphoreType.DMA,
            pltpu.SemaphoreType.DMA,
            pltpu.SemaphoreType.DMA,
            pltpu.SemaphoreType.DMA,
        ],
    )
    return k(x_flat, table)


def kernel(x, table):
    x_flat = x.reshape(-1).astype(jnp.int32)
    out = _gather(x_flat, table)
    return out.reshape(BATCH, HIST, EMB_DIM)
